# Initial kernel scaffold; baseline (speedup 1.0000x reference)
import jax
import jax.numpy as jnp
from jax.experimental import pallas as pl


def kernel(z_e, emb):
    raise NotImplementedError("write your pallas kernel here")


# speedup vs baseline: 1.1604x; 1.1604x over previous
import jax
import jax.numpy as jnp
from jax.experimental import pallas as pl

_CODEBOOK = 1024
_DIM = 64
_ROWS = 16384
_R = 512
_G = _ROWS // _R
_BETA = 0.25


def _vq_body(x_ref, sx_ref, se_ref, emb_ref, zq_ref, idx_ref, loss_ref):
    i = pl.program_id(0)
    x = x_ref[...]
    emb = emb_ref[...]
    t = jax.lax.dot_general(x, emb, (((1,), (1,)), ((), ())),
                            preferred_element_type=jnp.float32)
    d = (sx_ref[...] - 2.0 * t) + se_ref[...]
    m = jnp.min(d, axis=1, keepdims=True)
    lanes = jax.lax.broadcasted_iota(jnp.int32, d.shape, 1)
    idx = jnp.min(jnp.where(d == m, lanes, _CODEBOOK), axis=1, keepdims=True)
    idx_ref[...] = idx
    onehot = (lanes == idx).astype(jnp.float32)
    zq = jax.lax.dot_general(onehot, emb, (((1,), (0,)), ((), ())),
                             preferred_element_type=jnp.float32)
    zq_ref[...] = x + (zq - x)

    part = jnp.sum((zq - x) ** 2).reshape(1, 1)

    @pl.when(i == 0)
    def _():
        loss_ref[...] = jnp.zeros((1, 1), jnp.float32)

    loss_ref[...] += part

    @pl.when(i == _G - 1)
    def _():
        loss_ref[...] = loss_ref[...] * ((1.0 + _BETA) / float(_ROWS * _DIM))


def kernel(z_e, emb):
    bsz, num_slots, code_dim = z_e.shape
    flat = z_e.reshape(-1, code_dim).astype(jnp.float32)
    e = emb.astype(jnp.float32)
    sx = jnp.sum(flat ** 2, axis=1, keepdims=True)
    se = jnp.sum(e ** 2, axis=1, keepdims=True).T

    zq_st, idx, loss = pl.pallas_call(
        _vq_body,
        grid=(_G,),
        in_specs=[
            pl.BlockSpec((_R, _DIM), lambda i: (i, 0)),
            pl.BlockSpec((_R, 1), lambda i: (i, 0)),
            pl.BlockSpec((1, _CODEBOOK), lambda i: (0, 0)),
            pl.BlockSpec((_CODEBOOK, _DIM), lambda i: (0, 0)),
        ],
        out_specs=[
            pl.BlockSpec((_R, _DIM), lambda i: (i, 0)),
            pl.BlockSpec((_R, 1), lambda i: (i, 0)),
            pl.BlockSpec((1, 1), lambda i: (0, 0)),
        ],
        out_shape=[
            jax.ShapeDtypeStruct((_ROWS, _DIM), jnp.float32),
            jax.ShapeDtypeStruct((_ROWS, 1), jnp.int32),
            jax.ShapeDtypeStruct((1, 1), jnp.float32),
        ],
    )(flat, sx, se, e)

    return (zq_st.reshape(bsz, num_slots, code_dim),
            idx.reshape(bsz, num_slots),
            loss[0, 0])
